# baseline (speedup 1.0000x reference)
import jax
import jax.numpy as jnp
from jax.experimental import pallas as pl

G = 100


def _seg_mean(batch2d, xv):
    n = xv.shape[0]
    ids = jax.lax.broadcasted_iota(jnp.int32, (G, n), 0)
    seg = (batch2d == ids).astype(jnp.float32)
    s = jnp.dot(seg, xv, preferred_element_type=jnp.float32)
    cnt = jnp.sum(seg, axis=1, keepdims=True)
    return s / jnp.clip(cnt, 1.0)


def _mm_body(x_ref, w_ref, o_ref):
    o_ref[...] = jnp.dot(x_ref[...], w_ref[...], preferred_element_type=jnp.float32)


def _mm(x, w):
    return pl.pallas_call(
        _mm_body,
        out_shape=jax.ShapeDtypeStruct((x.shape[0], w.shape[1]), jnp.float32),
    )(x, w)


def _stage_body(a_ref, deg_ref, x_ref, wroot_ref, b_ref, wnext_ref, batch_ref,
                xo_ref, tnext_ref, pool_ref):
    y = (
        a_ref[...] / jnp.clip(deg_ref[...], 1.0)
        + b_ref[...]
        + jnp.dot(x_ref[...], wroot_ref[...], preferred_element_type=jnp.float32)
    )
    xo = jnp.maximum(y, 0.0)
    xo_ref[...] = xo
    tnext_ref[...] = jnp.dot(xo, wnext_ref[...], preferred_element_type=jnp.float32)
    pool_ref[...] = _seg_mean(batch_ref[...], xo)


def _stage(a_sum, deg, x, w_root, b, w_next, batch):
    n, h = a_sum.shape
    return pl.pallas_call(
        _stage_body,
        out_shape=(
            jax.ShapeDtypeStruct((n, h), jnp.float32),
            jax.ShapeDtypeStruct((n, w_next.shape[1]), jnp.float32),
            jax.ShapeDtypeStruct((G, h), jnp.float32),
        ),
    )(a_sum, deg[:, None], x, w_root, b[None, :], w_next, batch[None, :])


def _score_body(sp_ref, deg_ref, x_ref, wproot_ref, bp_ref, o_ref):
    o_ref[...] = jnp.tanh(
        sp_ref[...] / jnp.clip(deg_ref[...], 1.0)
        + bp_ref[...]
        + jnp.dot(x_ref[...], wproot_ref[...], preferred_element_type=jnp.float32)
    )


def _score(sp, deg, x2, wp_root, bp):
    n = x2.shape[0]
    wp8 = jnp.pad(wp_root, ((0, 0), (0, 7)))
    bp8 = jnp.pad(bp, (0, 7))
    out = pl.pallas_call(
        _score_body,
        out_shape=jax.ShapeDtypeStruct((n, 8), jnp.float32),
    )(sp, deg[:, None], x2, wp8, bp8[None, :])
    return out[:, 0]


def _final_body(a_ref, deg_ref, x_ref, wroot_ref, b_ref, batch_ref,
                xs0_ref, xs1_ref, w1_ref, b1_ref, w2_ref, b2_ref, o_ref):
    y = (
        a_ref[...] / jnp.clip(deg_ref[...], 1.0)
        + b_ref[...]
        + jnp.dot(x_ref[...], wroot_ref[...], preferred_element_type=jnp.float32)
    )
    x4 = jnp.maximum(y, 0.0)
    xs2 = _seg_mean(batch_ref[...], x4)
    h = jnp.concatenate([xs0_ref[...], xs1_ref[...], xs2], axis=1)
    t = jnp.maximum(
        jnp.dot(h, w1_ref[...], preferred_element_type=jnp.float32) + b1_ref[...],
        0.0,
    )
    z = jnp.dot(t, w2_ref[...], preferred_element_type=jnp.float32) + b2_ref[...]
    m = jnp.max(z, axis=-1, keepdims=True)
    o_ref[...] = z - m - jnp.log(jnp.sum(jnp.exp(z - m), axis=-1, keepdims=True))


def _final(a_sum, deg2, x3, w_root, b, batch2, xs0, xs1, w1, b1, w2, b2):
    return pl.pallas_call(
        _final_body,
        out_shape=jax.ShapeDtypeStruct((G, w2.shape[1]), jnp.float32),
    )(a_sum, deg2[:, None], x3, w_root, b[None, :], batch2[None, :],
      xs0, xs1, w1, b1[None, :], w2, b2[None, :])


def _scatter(vals, src, dst, n):
    return jnp.zeros((n, vals.shape[1]), vals.dtype).at[dst].add(vals[src])


def _topk_perm(score, batch):
    n = batch.shape[0]
    counts = jnp.zeros((G,), jnp.int32).at[batch].add(1)
    ptr = jnp.concatenate([jnp.zeros((1,), jnp.int32), jnp.cumsum(counts)])
    local = jnp.arange(n) - ptr[batch]
    dense = jnp.full((G, n), -jnp.inf, jnp.float32).at[batch, local].set(
        score.astype(jnp.float32)
    )
    order = jnp.argsort(-dense, axis=1)
    k = (4 * counts + 4) // 5
    node_idx = ptr[:G][:, None] + order
    sel = jnp.arange(n)[None, :] < k[:, None]
    ord_flat = jnp.argsort(jnp.logical_not(sel).reshape(-1).astype(jnp.int32))[:n]
    perm = node_idx.reshape(-1)[ord_flat].astype(jnp.int32)
    valid = sel.reshape(-1)[ord_flat]
    return perm, valid


def _filter_adj(ei, perm, valid, n):
    src, dst = ei[0], ei[1]
    np_ = perm.shape[0]
    node_mask = jnp.zeros((n,), jnp.int32).at[perm].add(valid.astype(jnp.int32)) > 0
    perm_safe = jnp.where(valid, perm, n)
    new_id = jnp.full((n,), -1, jnp.int32).at[perm_safe].set(
        jnp.arange(np_, dtype=jnp.int32)
    )
    em = node_mask[src] & node_mask[dst]
    new_src = jnp.where(em, new_id[src], 0)
    new_dst = jnp.where(em, new_id[dst], np_)
    return jnp.stack([new_src, new_dst])


def kernel(x, edge_index, batch, W1_rel, b1, W1_root, Wc0_rel, bc0, Wc0_root,
           Wp_rel, bp, Wp_root, Wc1_rel, bc1, Wc1_root, Wl1, bl1, Wl2, bl2):
    n = x.shape[0]
    src, dst = edge_index[0], edge_index[1]
    deg = jnp.zeros((n,), jnp.float32).at[dst].add(1.0)

    t1 = _mm(x, W1_rel)
    a1 = _scatter(t1, src, dst, n)
    x1, t2, xs0 = _stage(a1, deg, x, W1_root, b1, Wc0_rel, batch)

    a2 = _scatter(t2, src, dst, n)
    wp8 = jnp.pad(Wp_rel, ((0, 0), (0, 7)))
    x2, t3, xs1 = _stage(a2, deg, x1, Wc0_root, bc0, wp8, batch)

    sp = _scatter(t3[:, :1], src, dst, n)
    score = _score(sp, deg, x2, Wp_root, bp)

    perm, valid = _topk_perm(score, batch)
    x3 = x2[perm] * score[perm][:, None]
    batch2 = jnp.where(valid, batch[perm], G)
    ei2 = _filter_adj(edge_index, perm, valid, n)

    t4 = _mm(x3, Wc1_rel)
    src2, dst2 = ei2[0], ei2[1]
    a3 = _scatter(t4, src2, dst2, n)
    deg2 = jnp.zeros((n,), jnp.float32).at[dst2].add(1.0)

    return _final(a3, deg2, x3, Wc1_root, bc1, batch2, xs0, xs1,
                  Wl1, bl1, Wl2, bl2)


# speedup vs baseline: 1.1521x; 1.1521x over previous
import functools

import jax
import jax.numpy as jnp
from jax.experimental import pallas as pl

G = 100


def _gc_body(agg_ref, deg_ref, x_ref, wr_ref, b_ref, wroot_ref, o_ref, *, act):
    agg = agg_ref[...] / jnp.clip(deg_ref[...], 1.0)
    y = (
        jnp.dot(agg, wr_ref[...], preferred_element_type=jnp.float32)
        + b_ref[...]
        + jnp.dot(x_ref[...], wroot_ref[...], preferred_element_type=jnp.float32)
    )
    if act == "relu":
        y = jnp.maximum(y, 0.0)
    elif act == "tanh":
        y = jnp.tanh(y)
    o_ref[...] = y


def _gc(agg_sum, deg, x, w_rel, b, w_root, act):
    n = x.shape[0]
    h = w_rel.shape[1]
    return pl.pallas_call(
        functools.partial(_gc_body, act=act),
        out_shape=jax.ShapeDtypeStruct((n, h), jnp.float32),
    )(agg_sum, deg[:, None], x, w_rel, b[None, :], w_root)


def _gmp_body(x_ref, batch_ref, o_ref):
    n = x_ref.shape[0]
    ids = jax.lax.broadcasted_iota(jnp.int32, (G, n), 0)
    seg = (batch_ref[...] == ids).astype(jnp.float32)
    s = jnp.dot(seg, x_ref[...], preferred_element_type=jnp.float32)
    cnt = jnp.sum(seg, axis=1, keepdims=True)
    o_ref[...] = s / jnp.clip(cnt, 1.0)


def _gmp(x, batch):
    return pl.pallas_call(
        _gmp_body,
        out_shape=jax.ShapeDtypeStruct((G, x.shape[1]), jnp.float32),
    )(x, batch[None, :])


def _head_body(h_ref, w1_ref, b1_ref, w2_ref, b2_ref, o_ref):
    t = jnp.maximum(
        jnp.dot(h_ref[...], w1_ref[...], preferred_element_type=jnp.float32)
        + b1_ref[...],
        0.0,
    )
    z = jnp.dot(t, w2_ref[...], preferred_element_type=jnp.float32) + b2_ref[...]
    m = jnp.max(z, axis=-1, keepdims=True)
    e = jnp.exp(z - m)
    o_ref[...] = z - m - jnp.log(jnp.sum(e, axis=-1, keepdims=True))


def _head(h, w1, b1, w2, b2):
    return pl.pallas_call(
        _head_body,
        out_shape=jax.ShapeDtypeStruct((h.shape[0], w2.shape[1]), jnp.float32),
    )(h, w1, b1[None, :], w2, b2[None, :])


def _agg(x, ei):
    src, dst = ei[0], ei[1]
    n = x.shape[0]
    agg = jnp.zeros((n, x.shape[1]), x.dtype).at[dst].add(x[src])
    deg = jnp.zeros((n,), x.dtype).at[dst].add(1.0)
    return agg, deg


def _topk_perm(score, batch):
    n = batch.shape[0]
    counts = jnp.zeros((G,), jnp.int32).at[batch].add(1)
    ptr = jnp.concatenate([jnp.zeros((1,), jnp.int32), jnp.cumsum(counts)])
    k = (4 * counts + 4) // 5
    _, _, sidx = jax.lax.sort(
        (batch, -score.astype(jnp.float32), jnp.arange(n, dtype=jnp.int32)),
        num_keys=2,
        is_stable=True,
    )
    local = jnp.arange(n, dtype=jnp.int32) - ptr[batch]
    sel = local < k[batch]
    part = jnp.argsort(jnp.logical_not(sel))
    perm = sidx[part]
    valid = sel[part]
    return perm, valid


def _filter_adj(ei, perm, valid, n):
    src, dst = ei[0], ei[1]
    np_ = perm.shape[0]
    node_mask = jnp.zeros((n,), jnp.int32).at[perm].add(valid.astype(jnp.int32)) > 0
    perm_safe = jnp.where(valid, perm, n)
    new_id = jnp.full((n,), -1, jnp.int32).at[perm_safe].set(
        jnp.arange(np_, dtype=jnp.int32)
    )
    em = node_mask[src] & node_mask[dst]
    new_src = jnp.where(em, new_id[src], 0)
    new_dst = jnp.where(em, new_id[dst], np_)
    return jnp.stack([new_src, new_dst])


def kernel(x, edge_index, batch, W1_rel, b1, W1_root, Wc0_rel, bc0, Wc0_root,
           Wp_rel, bp, Wp_root, Wc1_rel, bc1, Wc1_root, Wl1, bl1, Wl2, bl2):
    n = x.shape[0]

    agg1, deg = _agg(x, edge_index)
    x1 = _gc(agg1, deg, x, W1_rel, b1, W1_root, "relu")
    xs0 = _gmp(x1, batch)

    agg2, _ = _agg(x1, edge_index)
    x2 = _gc(agg2, deg, x1, Wc0_rel, bc0, Wc0_root, "relu")
    xs1 = _gmp(x2, batch)

    aggp, _ = _agg(x2, edge_index)
    wp_rel = jnp.pad(Wp_rel, ((0, 0), (0, 7)))
    wp_root = jnp.pad(Wp_root, ((0, 0), (0, 7)))
    bp_p = jnp.pad(bp, (0, 7))
    score = _gc(aggp, deg, x2, wp_rel, bp_p, wp_root, "tanh")[:, 0]

    perm, valid = _topk_perm(score, batch)
    x3 = x2[perm] * score[perm][:, None]
    batch2 = jnp.where(valid, batch[perm], G)
    ei2 = _filter_adj(edge_index, perm, valid, n)

    agg3, deg2 = _agg(x3, ei2)
    x4 = _gc(agg3, deg2, x3, Wc1_rel, bc1, Wc1_root, "relu")
    xs2 = _gmp(x4, batch2)

    h = jnp.concatenate([xs0, xs1, xs2], axis=1)
    return _head(h, Wl1, bl1, Wl2, bl2)
